# baseline (speedup 1.0000x reference)
import functools

import jax
import jax.numpy as jnp
from jax import lax
from jax.experimental import pallas as pl
from jax.experimental.pallas import tpu as pltpu
from jax.experimental.pallas import tpu_sc as plsc

EMBED_DIM = 32
NUM_CORES = 2
NUM_SUBCORES = 16
NUM_WORKERS = NUM_CORES * NUM_SUBCORES

N_TOTAL = 8 * 20 * 64 * 64
B_PER_W = N_TOTAL // NUM_WORKERS
CHUNK = 1024
N_CHUNKS = B_PER_W // CHUNK
N_SLOTS = 3


def _sc_gather(x_flat, table):
  mesh = plsc.VectorSubcoreMesh(
      core_axis_name="c", subcore_axis_name="s",
      num_cores=NUM_CORES, num_subcores=NUM_SUBCORES)

  @functools.partial(
      pl.kernel,
      mesh=mesh,
      out_type=jax.ShapeDtypeStruct((N_TOTAL, EMBED_DIM), jnp.float32),
      scratch_types=[
          pltpu.VMEM((N_CHUNKS, CHUNK), jnp.int32),
          pltpu.VMEM((N_SLOTS, CHUNK, EMBED_DIM), jnp.float32),
          pltpu.SemaphoreType.DMA((N_SLOTS,)),
          pltpu.SemaphoreType.DMA((N_SLOTS,)),
      ],
      compiler_params=pltpu.CompilerParams(use_tc_tiling_on_sc=False),
  )
  def k(x_hbm, table_hbm, out_hbm, idx_v, rows_v, gsem, osem):
    wid = lax.axis_index("s") * NUM_CORES + lax.axis_index("c")
    base = wid * B_PER_W

    pltpu.sync_copy(x_hbm.at[wid], idx_v)

    def start_gather(i):
      s = i % N_SLOTS
      return pltpu.async_copy(
          table_hbm.at[idx_v.at[i]], rows_v.at[s], gsem.at[s])

    def start_out(i):
      s = i % N_SLOTS
      return pltpu.async_copy(
          rows_v.at[s], out_hbm.at[pl.ds(base + i * CHUNK, CHUNK)],
          osem.at[s])

    gathers = {0: start_gather(0), 1: start_gather(1)}
    outs = {}
    for i in range(N_CHUNKS):
      gathers.pop(i).wait()
      outs[i] = start_out(i)
      if i + 2 < N_CHUNKS:
        if i - 1 in outs:
          outs.pop(i - 1).wait()
        gathers[i + 2] = start_gather(i + 2)
    for i in sorted(outs):
      outs.pop(i).wait()

  return k(x_flat, table)


def kernel(x, table):
  x3 = x.reshape(NUM_WORKERS, N_CHUNKS, CHUNK)
  out = _sc_gather(x3, table)
  return out.reshape(x.shape + (EMBED_DIM,))


# speedup vs baseline: 1.3071x; 1.3071x over previous
import functools

import jax
import jax.numpy as jnp
from jax import lax
from jax.experimental import pallas as pl
from jax.experimental.pallas import tpu as pltpu
from jax.experimental.pallas import tpu_sc as plsc

N_ROWS = 1000
EMBED_DIM = 32
NUM_CORES = 2
NUM_SUBCORES = 16
NUM_WORKERS = NUM_CORES * NUM_SUBCORES

N_TOTAL = 8 * 20 * 64 * 64
B_PER_W = N_TOTAL // NUM_WORKERS
CHUNK = 1024
N_CHUNKS = B_PER_W // CHUNK
N_SLOTS = 3


def _sc_gather(x_flat, table):
  mesh = plsc.VectorSubcoreMesh(
      core_axis_name="c", subcore_axis_name="s",
      num_cores=NUM_CORES, num_subcores=NUM_SUBCORES)

  @functools.partial(
      pl.kernel,
      mesh=mesh,
      out_type=jax.ShapeDtypeStruct((N_TOTAL, EMBED_DIM), jnp.float32),
      scratch_types=[
          pltpu.VMEM_SHARED((N_ROWS, EMBED_DIM), jnp.float32),
          pltpu.VMEM((N_CHUNKS, CHUNK), jnp.int32),
          pltpu.VMEM((N_SLOTS, CHUNK, EMBED_DIM), jnp.float32),
          pltpu.SemaphoreType.DMA((N_SLOTS,)),
          pltpu.SemaphoreType.DMA((N_SLOTS,)),
      ],
      compiler_params=pltpu.CompilerParams(use_tc_tiling_on_sc=False),
  )
  def k(x_hbm, table_hbm, out_hbm, table_sh, idx_v, rows_v, gsem, osem):
    wid = lax.axis_index("s") * NUM_CORES + lax.axis_index("c")
    base = wid * B_PER_W

    @pl.when(lax.axis_index("s") == 0)
    def _():
      pltpu.sync_copy(table_hbm, table_sh)

    pltpu.sync_copy(x_hbm.at[wid], idx_v)
    plsc.subcore_barrier()

    def start_gather(i):
      s = i % N_SLOTS
      return pltpu.async_copy(
          table_sh.at[idx_v.at[i]], rows_v.at[s], gsem.at[s])

    def start_out(i):
      s = i % N_SLOTS
      return pltpu.async_copy(
          rows_v.at[s], out_hbm.at[pl.ds(base + i * CHUNK, CHUNK)],
          osem.at[s])

    gathers = {0: start_gather(0), 1: start_gather(1)}
    outs = {}
    for i in range(N_CHUNKS):
      gathers.pop(i).wait()
      outs[i] = start_out(i)
      if i + 2 < N_CHUNKS:
        if i - 1 in outs:
          outs.pop(i - 1).wait()
        gathers[i + 2] = start_gather(i + 2)
    for i in sorted(outs):
      outs.pop(i).wait()

  return k(x_flat, table)


def kernel(x, table):
  x3 = x.reshape(NUM_WORKERS, N_CHUNKS, CHUNK)
  out = _sc_gather(x3, table)
  return out.reshape(x.shape + (EMBED_DIM,))


# speedup vs baseline: 1.3110x; 1.0030x over previous
import functools

import jax
import jax.numpy as jnp
from jax import lax
from jax.experimental import pallas as pl
from jax.experimental.pallas import tpu as pltpu
from jax.experimental.pallas import tpu_sc as plsc

N_ROWS = 1000
EMBED_DIM = 32
NUM_CORES = 2
NUM_SUBCORES = 16
NUM_WORKERS = NUM_CORES * NUM_SUBCORES

B, T, H, W = 8, 20, 64, 64
T_PER_W = (B * T) // NUM_WORKERS
STRIPE = 16
N_STRIPES = H // STRIPE
N_CHUNKS = T_PER_W * N_STRIPES
N_SLOTS = 3


def _sc_gather(x, table):
  mesh = plsc.VectorSubcoreMesh(
      core_axis_name="c", subcore_axis_name="s",
      num_cores=NUM_CORES, num_subcores=NUM_SUBCORES)

  @functools.partial(
      pl.kernel,
      mesh=mesh,
      out_type=jax.ShapeDtypeStruct((B, T, H, W, EMBED_DIM), jnp.float32),
      scratch_types=[
          pltpu.VMEM_SHARED((N_ROWS, EMBED_DIM), jnp.float32),
          pltpu.VMEM((N_SLOTS, STRIPE, W), jnp.int32),
          pltpu.VMEM((N_SLOTS, STRIPE, W, EMBED_DIM), jnp.float32),
          pltpu.SemaphoreType.DMA((N_SLOTS,)),
          pltpu.SemaphoreType.DMA((N_SLOTS,)),
          pltpu.SemaphoreType.DMA((N_SLOTS,)),
      ],
      compiler_params=pltpu.CompilerParams(use_tc_tiling_on_sc=False),
  )
  def k(x_hbm, table_hbm, out_hbm, table_sh, idx_v, rows_v, isem, gsem, osem):
    wid = lax.axis_index("s") * NUM_CORES + lax.axis_index("c")
    b = wid // (NUM_WORKERS // B)
    t0 = (wid % (NUM_WORKERS // B)) * T_PER_W

    @pl.when(lax.axis_index("s") == 0)
    def _():
      pltpu.sync_copy(table_hbm, table_sh)

    def start_idx(i):
      s = i % N_SLOTS
      p, q = divmod(i, N_STRIPES)
      return pltpu.async_copy(
          x_hbm.at[b, t0 + p, pl.ds(q * STRIPE, STRIPE)],
          idx_v.at[s], isem.at[s])

    def start_gathers(i):
      s = i % N_SLOTS
      return [
          pltpu.async_copy(
              table_sh.at[idx_v.at[s, r]], rows_v.at[s, r], gsem.at[s])
          for r in range(STRIPE)
      ]

    def start_out(i):
      s = i % N_SLOTS
      p, q = divmod(i, N_STRIPES)
      return pltpu.async_copy(
          rows_v.at[s],
          out_hbm.at[b, t0 + p, pl.ds(q * STRIPE, STRIPE)],
          osem.at[s])

    idxs = {0: start_idx(0), 1: start_idx(1)}
    plsc.subcore_barrier()

    idxs[0].wait()
    gathers = {0: start_gathers(0)}
    outs = {}
    for i in range(N_CHUNKS):
      if i + 2 < N_CHUNKS:
        idxs[i + 2] = start_idx(i + 2)
      if i + 1 < N_CHUNKS:
        if i - 2 in outs:
          outs.pop(i - 2).wait()
        idxs.pop(i + 1).wait()
        gathers[i + 1] = start_gathers(i + 1)
      for g in gathers.pop(i):
        g.wait()
      outs[i] = start_out(i)
    for i in sorted(outs):
      outs.pop(i).wait()

  return k(x, table)


def kernel(x, table):
  return _sc_gather(x, table)
